# baseline (speedup 1.0000x reference)
import jax
import jax.numpy as jnp
from jax.experimental import pallas as pl


def _pos_kernel(row_ref, col_ref, out_ref):
    b, twod, hw = out_ref.shape
    d = twod // 2
    h = row_ref.shape[0]
    w = col_ref.shape[0]
    colT = col_ref[...].T
    rowT = row_ref[...].T
    xe = jnp.broadcast_to(colT.reshape(d, 1, w), (d, h, w)).reshape(d, hw)
    ye = jnp.broadcast_to(rowT.reshape(d, h, 1), (d, h, w)).reshape(d, hw)
    pos = jnp.concatenate([xe, ye], axis=0)
    out_ref[...] = jnp.broadcast_to(pos[None], (b, twod, hw))


def kernel(input_tensor, row_embed, col_embed):
    b = input_tensor.shape[0]
    h, w = input_tensor.shape[-2], input_tensor.shape[-1]
    d = row_embed.shape[-1]
    row = jax.lax.slice(row_embed, (0, 0), (h, d))
    col = jax.lax.slice(col_embed, (0, 0), (w, d))
    out = pl.pallas_call(
        _pos_kernel,
        out_shape=jax.ShapeDtypeStruct((b, 2 * d, h * w), row_embed.dtype),
    )(row, col)
    return out.reshape(b, 2 * d, h, w)
